# baseline (speedup 1.0000x reference)
import functools

import numpy as np
import jax
import jax.numpy as jnp
from jax import lax
from jax.experimental import pallas as pl
from jax.experimental.pallas import tpu as pltpu
from jax.experimental.pallas import tpu_sc as plsc

_LANES = 16


def _positional_encoding_np(seq_len, d_model):
    pos = np.arange(seq_len, dtype=np.float32)[:, None]
    i = np.arange(0, d_model, 2, dtype=np.float32)[None, :]
    angles = pos / np.power(10000.0, i / d_model)
    pe = np.zeros((seq_len, d_model), dtype=np.float32)
    pe[:, 0::2] = np.sin(angles)
    pe[:, 1::2] = np.cos(angles)
    return pe


@functools.lru_cache(maxsize=None)
def _build(B, L, D, V):
    info = plsc.get_sparse_core_info()
    NC, NS = info.num_cores, info.num_subcores
    NW = NC * NS
    assert L % 8 == 0 and B % 128 == 0 and D % 8 == 0
    LT, BT, DT = L // 8, B // 128, D // 8
    assert BT % 2 == 0 and NW % (BT // 2) == 0
    btp_per = BT // 2
    l_splits = NW // btp_per
    assert L % l_splits == 0
    l_per = L // l_splits
    RB = 2

    mesh = plsc.VectorSubcoreMesh(core_axis_name="c", subcore_axis_name="s")

    @functools.partial(
        pl.kernel,
        mesh=mesh,
        compiler_params=pltpu.CompilerParams(use_tc_tiling_on_sc=False,
                                             needs_layout_passes=False),
        out_type=jax.ShapeDtypeStruct((L, DT, BT, 8, 128), jnp.float32),
        scratch_types=(
            [pltpu.VMEM((256,), jnp.int32) for _ in range(RB)]
            + [pltpu.VMEM((256, D), jnp.float32) for _ in range(RB)]
            + [pltpu.VMEM((DT, 2, 8, 128), jnp.float32) for _ in range(RB)]
            + [pltpu.VMEM((L, D), jnp.float32)]
            + [pltpu.SemaphoreType.DMA for _ in range(2 * RB)]
        ),
    )
    def _k(x4_hbm, pe_hbm, table_hbm, out5_hbm, *scratch):
        idx_v = scratch[:RB]
        rows_v = scratch[RB:2 * RB]
        outt_v = scratch[2 * RB:3 * RB]
        pe_v = scratch[3 * RB]
        gsem = scratch[3 * RB + 1:3 * RB + 1 + RB]
        osem = scratch[3 * RB + 1 + RB:]

        wid = lax.axis_index("s") * NC + lax.axis_index("c")
        btp = wid % btp_per
        l0 = (wid // btp_per) * l_per
        pltpu.sync_copy(pe_hbm, pe_v)

        def fetch(g, r):
            l = l0 + g
            lt = l // 8
            lr = l % 8
            for h in range(2):
                pltpu.sync_copy(x4_hbm.at[lt, 2 * btp + h, lr],
                                idx_v[r].at[pl.ds(128 * h, 128)])
            pltpu.async_copy(table_hbm.at[idx_v[r]], rows_v[r], gsem[r])

        fetch(0, 0)

        @pl.loop(0, l_per)
        def _grp(g):
            r = lax.rem(g, RB)

            @pl.when(g + 1 < l_per)
            def _():
                for rr in range(RB):
                    @pl.when(lax.rem(g + 1, RB) == rr)
                    def _():
                        @pl.when(g + 1 >= RB)
                        def _():
                            l_old = l0 + g + 1 - RB
                            pltpu.make_async_copy(
                                outt_v[rr],
                                out5_hbm.at[l_old, :,
                                            pl.ds(2 * btp, 2)],
                                osem[rr]).wait()
                        fetch(g + 1, rr)

            for rr in range(RB):
                @pl.when(r == rr)
                def _():
                    l = l0 + g
                    pltpu.make_async_copy(table_hbm.at[idx_v[rr]],
                                          rows_v[rr], gsem[rr]).wait()
                    lane = lax.iota(jnp.int32, 16)

                    @pl.loop(0, DT)
                    def _dt(dt):
                        for dr in range(8):
                            d = dt * 8 + dr
                            dvec = jnp.full((16,), d, jnp.int32)
                            pe_s = plsc.load_gather(
                                pe_v, [jnp.full((16,), l, jnp.int32),
                                       dvec])
                            for h in range(2):
                                for bg in range(8):
                                    rr_idx = lane + (h * 128 + bg * 16)
                                    vals = plsc.load_gather(
                                        rows_v[rr], [rr_idx, dvec])
                                    outt_v[rr][dt, h, dr,
                                               pl.ds(bg * 16, 16)] = (
                                        vals + pe_s)
                    pltpu.async_copy(
                        outt_v[rr],
                        out5_hbm.at[l, :, pl.ds(2 * btp, 2)],
                        osem[rr])

        for rr in range(RB):
            g_last = l_per - RB + rr
            pltpu.make_async_copy(
                outt_v[rr],
                out5_hbm.at[l0 + g_last, :, pl.ds(2 * btp, 2)],
                osem[rr]).wait()

    return _k


def kernel(x, table):
    B, L = x.shape
    V, D = table.shape
    pe = jnp.asarray(_positional_encoding_np(L, D))
    x4 = x.T.reshape(L // 8, 8, B // 128, 128).transpose(0, 2, 1, 3)
    x4 = x4.astype(jnp.int32)
    out5 = _build(B, L, D, V)(x4, pe, table)
    out = out5.transpose(2, 4, 0, 1, 3).reshape(B, L, D)
    return out


# speedup vs baseline: 1.3080x; 1.3080x over previous
import functools

import numpy as np
import jax
import jax.numpy as jnp
from jax import lax
from jax.experimental import pallas as pl
from jax.experimental.pallas import tpu as pltpu
from jax.experimental.pallas import tpu_sc as plsc

_LANES = 16


def _positional_encoding_np(seq_len, d_model):
    pos = np.arange(seq_len, dtype=np.float32)[:, None]
    i = np.arange(0, d_model, 2, dtype=np.float32)[None, :]
    angles = pos / np.power(10000.0, i / d_model)
    pe = np.zeros((seq_len, d_model), dtype=np.float32)
    pe[:, 0::2] = np.sin(angles)
    pe[:, 1::2] = np.cos(angles)
    return pe


@functools.lru_cache(maxsize=None)
def _build(B, L, D, V):
    info = plsc.get_sparse_core_info()
    NC, NS = info.num_cores, info.num_subcores
    NW = NC * NS
    assert L % 8 == 0 and B % 128 == 0 and D % 8 == 0
    LT, BT, DT = L // 8, B // 128, D // 8
    assert BT % 2 == 0 and NW % (BT // 2) == 0
    btp_per = BT // 2
    l_splits = NW // btp_per
    assert L % l_splits == 0
    l_per = L // l_splits
    RB = 2
    DG = D // _LANES

    mesh = plsc.VectorSubcoreMesh(core_axis_name="c", subcore_axis_name="s")

    @functools.partial(
        pl.kernel,
        mesh=mesh,
        compiler_params=pltpu.CompilerParams(use_tc_tiling_on_sc=False,
                                             needs_layout_passes=False),
        out_type=jax.ShapeDtypeStruct((L, DT, BT, 8, 128), jnp.float32),
        scratch_types=(
            [pltpu.VMEM((2, 128), jnp.int32) for _ in range(RB)]
            + [pltpu.VMEM((2, 128, D), jnp.float32) for _ in range(RB)]
            + [pltpu.VMEM((DT, 2, 8, 128), jnp.float32) for _ in range(RB)]
            + [pltpu.VMEM((256 * (D + 1),), jnp.float32)]
            + [pltpu.VMEM((L, D), jnp.float32)]
            + [pltpu.SemaphoreType.DMA for _ in range(2 * RB)]
        ),
    )
    def _k(x4_hbm, pe_hbm, table_hbm, out5_hbm, *scratch):
        idx_v = scratch[:RB]
        rows_v = scratch[RB:2 * RB]
        outt_v = scratch[2 * RB:3 * RB]
        pad_v = scratch[3 * RB]
        pe_v = scratch[3 * RB + 1]
        gsem = scratch[3 * RB + 2:3 * RB + 2 + RB]
        osem = scratch[3 * RB + 2 + RB:]

        wid = lax.axis_index("s") * NC + lax.axis_index("c")
        btp = wid % btp_per
        l0 = (wid // btp_per) * l_per
        pltpu.sync_copy(pe_hbm, pe_v)

        def fetch(g, r):
            l = l0 + g
            lt = l // 8
            lr = l % 8
            for h in range(2):
                pltpu.sync_copy(x4_hbm.at[lt, 2 * btp + h, lr],
                                idx_v[r].at[h])
            for h in range(2):
                pltpu.async_copy(
                    table_hbm.at[idx_v[r].at[h]],
                    rows_v[r].at[h], gsem[r])

        fetch(0, 0)

        @pl.loop(0, l_per)
        def _grp(g):
            r = lax.rem(g, RB)

            @pl.when(g + 1 < l_per)
            def _():
                for rr in range(RB):
                    @pl.when(lax.rem(g + 1, RB) == rr)
                    def _():
                        fetch(g + 1, rr)

            for rr in range(RB):
                @pl.when(r == rr)
                def _():
                    l = l0 + g
                    for h in range(2):
                        pltpu.make_async_copy(
                            table_hbm.at[idx_v[rr].at[h]],
                            rows_v[rr].at[h], gsem[rr]).wait()

                    lane = lax.iota(jnp.int32, _LANES)
                    pes = [pe_v[l, pl.ds(dg * _LANES, _LANES)]
                           for dg in range(DG)]
                    lane_dg = [lane + dg * _LANES for dg in range(DG)]

                    for h in range(2):
                        @pl.loop(0, 128)
                        def _row(b):
                            base = jnp.full((16,), (h * 128 + b) * (D + 1),
                                            jnp.int32)
                            for dg in range(DG):
                                s = pl.ds(dg * _LANES, _LANES)
                                plsc.store_scatter(
                                    pad_v, [base + lane_dg[dg]],
                                    rows_v[rr][h, b, s] + pes[dg])

                    @pl.when(g >= RB)
                    def _():
                        l_old = l0 + g - RB
                        pltpu.make_async_copy(
                            outt_v[rr],
                            out5_hbm.at[l_old, :, pl.ds(2 * btp, 2)],
                            osem[rr]).wait()

                    lane65 = lane * (D + 1)

                    @pl.loop(0, DT)
                    def _dt(dt):
                        for dr in range(8):
                            d = dt * 8 + dr
                            for h in range(2):
                                for bg in range(8):
                                    base = (h * 128 + bg * 16) * (D + 1) + d
                                    vals = plsc.load_gather(
                                        pad_v,
                                        [lane65 + base])
                                    outt_v[rr][dt, h, dr,
                                               pl.ds(bg * 16, 16)] = vals

                    pltpu.async_copy(
                        outt_v[rr],
                        out5_hbm.at[l, :, pl.ds(2 * btp, 2)],
                        osem[rr])

        for rr in range(RB):
            g_last = l_per - RB + rr
            pltpu.make_async_copy(
                outt_v[rr],
                out5_hbm.at[l0 + g_last, :, pl.ds(2 * btp, 2)],
                osem[rr]).wait()

    return _k


def kernel(x, table):
    B, L = x.shape
    V, D = table.shape
    pe = jnp.asarray(_positional_encoding_np(L, D))
    x4 = x.T.reshape(L // 8, 8, B // 128, 128).transpose(0, 2, 1, 3)
    x4 = x4.astype(jnp.int32)
    out5 = _build(B, L, D, V)(x4, pe, table)
    out = out5.transpose(2, 4, 0, 1, 3).reshape(B, L, D)
    return out


# speedup vs baseline: 2.2155x; 1.6938x over previous
import functools

import numpy as np
import jax
import jax.numpy as jnp
from jax import lax
from jax.experimental import pallas as pl
from jax.experimental.pallas import tpu as pltpu
from jax.experimental.pallas import tpu_sc as plsc

_LANES = 16


def _positional_encoding_np(seq_len, d_model):
    pos = np.arange(seq_len, dtype=np.float32)[:, None]
    i = np.arange(0, d_model, 2, dtype=np.float32)[None, :]
    angles = pos / np.power(10000.0, i / d_model)
    pe = np.zeros((seq_len, d_model), dtype=np.float32)
    pe[:, 0::2] = np.sin(angles)
    pe[:, 1::2] = np.cos(angles)
    return pe


@functools.lru_cache(maxsize=None)
def _build(B, L, D, V):
    info = plsc.get_sparse_core_info()
    NC, NS = info.num_cores, info.num_subcores
    NW = NC * NS
    assert L % 8 == 0 and B % 128 == 0 and D % 8 == 0
    LT, BT, DT = L // 8, B // 128, D // 8
    assert BT % 2 == 0 and NW % (BT // 2) == 0
    btp_per = BT // 2
    l_splits = NW // btp_per
    assert L % l_splits == 0
    l_per = L // l_splits
    RB = 2
    DG = D // _LANES

    mesh = plsc.VectorSubcoreMesh(core_axis_name="c", subcore_axis_name="s")

    @functools.partial(
        pl.kernel,
        mesh=mesh,
        compiler_params=pltpu.CompilerParams(use_tc_tiling_on_sc=False,
                                             needs_layout_passes=False),
        out_type=jax.ShapeDtypeStruct((L, DT, BT, 8, 128), jnp.float32),
        scratch_types=(
            [pltpu.VMEM((2, 128), jnp.int32) for _ in range(RB)]
            + [pltpu.VMEM((2, 128, D), jnp.float32) for _ in range(RB)]
            + [pltpu.VMEM((DT, 2, 8, 128), jnp.float32) for _ in range(RB)]
            + [pltpu.VMEM((256 * (D + 1),), jnp.float32)]
            + [pltpu.VMEM((L, D), jnp.float32)]
            + [pltpu.SemaphoreType.DMA for _ in range(2 * RB)]
        ),
    )
    def _k(x4_hbm, pe_hbm, table_hbm, out5_hbm, *scratch):
        idx_v = scratch[:RB]
        rows_v = scratch[RB:2 * RB]
        outt_v = scratch[2 * RB:3 * RB]
        pad_v = scratch[3 * RB]
        pe_v = scratch[3 * RB + 1]
        gsem = scratch[3 * RB + 2:3 * RB + 2 + RB]
        osem = scratch[3 * RB + 2 + RB:]

        wid = lax.axis_index("s") * NC + lax.axis_index("c")
        btp = wid % btp_per
        l0 = (wid // btp_per) * l_per
        pltpu.sync_copy(pe_hbm, pe_v)

        def fetch(g, r):
            l = l0 + g
            lt = l // 8
            lr = l % 8
            for h in range(2):
                pltpu.sync_copy(x4_hbm.at[lt, 2 * btp + h, lr],
                                idx_v[r].at[h])
            for h in range(2):
                pltpu.async_copy(
                    table_hbm.at[idx_v[r].at[h]],
                    rows_v[r].at[h], gsem[r])

        fetch(0, 0)

        @pl.loop(0, l_per)
        def _grp(g):
            r = lax.rem(g, RB)

            @pl.when(g + 1 < l_per)
            def _():
                for rr in range(RB):
                    @pl.when(lax.rem(g + 1, RB) == rr)
                    def _():
                        fetch(g + 1, rr)

            for rr in range(RB):
                @pl.when(r == rr)
                def _():
                    l = l0 + g
                    for h in range(2):
                        pltpu.make_async_copy(
                            table_hbm.at[idx_v[rr].at[h]],
                            rows_v[rr].at[h], gsem[rr]).wait()

                    lane = lax.iota(jnp.int32, _LANES)
                    pes = [pe_v[l, pl.ds(dg * _LANES, _LANES)]
                           for dg in range(DG)]
                    lane_dg = [lane + dg * _LANES for dg in range(DG)]

                    for h in range(2):
                        @pl.loop(0, 128, step=2)
                        def _row(b):
                            bases = [jnp.full((16,),
                                              (h * 128 + b + u) * (D + 1),
                                              jnp.int32) for u in range(2)]
                            idxs = [bases[u] + lane_dg[dg]
                                    for u in range(2) for dg in range(DG)]
                            lds = [rows_v[rr][h, b + u,
                                              pl.ds(dg * _LANES, _LANES)]
                                   for u in range(2) for dg in range(DG)]
                            sums = [lds[u * DG + dg] + pes[dg]
                                    for u in range(2) for dg in range(DG)]
                            for k in range(2 * DG):
                                plsc.store_scatter(pad_v, [idxs[k]],
                                                   sums[k])

                    @pl.when(g >= RB)
                    def _():
                        l_old = l0 + g - RB
                        pltpu.make_async_copy(
                            outt_v[rr],
                            out5_hbm.at[l_old, :, pl.ds(2 * btp, 2)],
                            osem[rr]).wait()

                    lane65 = lane * (D + 1)

                    @pl.loop(0, DT)
                    def _dt(dt):
                        d0 = dt * 8
                        for h in range(2):
                            for bg in range(8):
                                base = lane65 + ((h * 128 + bg * 16)
                                                 * (D + 1) + d0)
                                idxs = [base + dr for dr in range(8)]
                                vals = [plsc.load_gather(pad_v, [ix])
                                        for ix in idxs]
                                for dr in range(8):
                                    outt_v[rr][dt, h, dr,
                                               pl.ds(bg * 16, 16)] = (
                                        vals[dr])

                    pltpu.async_copy(
                        outt_v[rr],
                        out5_hbm.at[l, :, pl.ds(2 * btp, 2)],
                        osem[rr])

        for rr in range(RB):
            g_last = l_per - RB + rr
            pltpu.make_async_copy(
                outt_v[rr],
                out5_hbm.at[l0 + g_last, :, pl.ds(2 * btp, 2)],
                osem[rr]).wait()

    return _k


def kernel(x, table):
    B, L = x.shape
    V, D = table.shape
    pe = jnp.asarray(_positional_encoding_np(L, D))
    x4 = x.T.reshape(L // 8, 8, B // 128, 128).transpose(0, 2, 1, 3)
    x4 = x4.astype(jnp.int32)
    out5 = _build(B, L, D, V)(x4, pe, table)
    out = out5.transpose(2, 4, 0, 1, 3).reshape(B, L, D)
    return out
